# baseline (speedup 1.0000x reference)
import functools

import jax
import jax.numpy as jnp
from jax import lax
from jax.experimental import pallas as pl


def _mm_body(x_ref, w_ref, b_ref, o_ref, *, act):
    acc = jnp.dot(x_ref[...], w_ref[...], preferred_element_type=jnp.float32)
    acc = acc + b_ref[...]
    if act == "relu":
        acc = jnp.maximum(acc, 0.0)
    elif act == "sigmoid":
        acc = jax.nn.sigmoid(acc)
    o_ref[...] = acc


@functools.partial(jax.jit, static_argnames=("act", "tr"))
def pallas_mm(x, w, b, act=None, tr=2000):
    n, k = x.shape
    m = w.shape[1]
    if n % tr:
        tr = n
    grid = (n // tr,)
    return pl.pallas_call(
        functools.partial(_mm_body, act=act),
        grid=grid,
        in_specs=[
            pl.BlockSpec((tr, k), lambda i: (i, 0)),
            pl.BlockSpec((k, m), lambda i: (0, 0)),
            pl.BlockSpec((1, m), lambda i: (0, 0)),
        ],
        out_specs=pl.BlockSpec((tr, m), lambda i: (i, 0)),
        out_shape=jax.ShapeDtypeStruct((n, m), jnp.float32),
    )(x, w, b.reshape(1, m))


def _greedy_merge(src, dst, scores, active):
    N = active.shape[0]
    E = src.shape[0]
    valid = src < N
    s_c = jnp.minimum(src, N - 1)
    t_c = jnp.minimum(dst, N - 1)

    order = jnp.argsort(jnp.where(valid, -scores, jnp.inf), stable=True)
    rank = jnp.zeros((E,), jnp.int32).at[order].set(jnp.arange(E, dtype=jnp.int32))

    INF = jnp.int32(2**30)
    alive0 = valid & active[s_c] & active[t_c]

    def cond(st):
        return st[0].any()

    def body(st):
        alive, matched, accept = st
        rk = jnp.where(alive, rank, INF)
        best = jnp.full((N,), INF, jnp.int32)
        best = best.at[s_c].min(rk)
        best = best.at[t_c].min(rk)
        acc = alive & (best[s_c] == rank) & (best[t_c] == rank)
        matched = matched.at[s_c].max(acc)
        matched = matched.at[t_c].max(acc)
        alive = alive & ~matched[s_c] & ~matched[t_c]
        return alive, matched, accept | acc

    matched0 = jnp.zeros((N,), bool)
    accept0 = jnp.zeros((E,), bool)
    alive, matched, accept = lax.while_loop(cond, body, (alive0, matched0, accept0))

    i = accept.sum().astype(jnp.int32)
    acc_by_rank = accept[order]
    cid_by_rank = jnp.cumsum(acc_by_rank.astype(jnp.int32)) - 1
    cid = jnp.zeros((E,), jnp.int32).at[order].set(cid_by_rank)

    cluster = jnp.full((N,), N, jnp.int32)
    cluster = cluster.at[jnp.where(accept, s_c, N)].set(jnp.where(accept, cid, N))
    cluster = cluster.at[jnp.where(accept, t_c, N)].set(jnp.where(accept, cid, N))
    nes = jnp.ones((N,), scores.dtype).at[jnp.where(accept, cid, N)].set(scores)

    remaining = active & ~matched
    cs = jnp.cumsum(remaining.astype(jnp.int32))
    cluster = jnp.where(remaining, i + cs - 1, cluster)
    return cluster, nes, i + cs[-1]


def _edge_scores(a, bb, src, dst, b0, N):
    s_c = jnp.minimum(src, N - 1)
    d_c = jnp.minimum(dst, N - 1)
    raw = a[s_c] + bb[d_c] + b0
    d_seg = jnp.where(src < N, dst, N)
    mx = jnp.full((N,), -jnp.inf, raw.dtype).at[d_seg].max(raw)
    ex = jnp.exp(raw - mx[d_c])
    den = jnp.zeros((N,), raw.dtype).at[d_seg].add(ex)
    return ex / den[d_c] + 0.5


def _coarsen_edges(cluster, src, dst, ncl, N, E):
    sent = jnp.int32(N * N)
    s_c = jnp.minimum(src, N - 1)
    d_c = jnp.minimum(dst, N - 1)
    key = jnp.where(src < N, cluster[s_c] * ncl + cluster[d_c], sent)
    keys = jnp.sort(key)
    uniq = jnp.concatenate([jnp.ones((1,), bool), keys[1:] != keys[:-1]]) & (keys < sent)
    pos = jnp.cumsum(uniq.astype(jnp.int32)) - 1
    out = jnp.full((E,), sent, keys.dtype).at[jnp.where(uniq, pos, E)].set(keys)
    ns = jnp.where(out < sent, out // ncl, N).astype(jnp.int32)
    nd = jnp.where(out < sent, out % ncl, N).astype(jnp.int32)
    return ns, nd


def _edge_pool(x, a, bb, src, dst, b0, active):
    N = x.shape[0]
    E = src.shape[0]
    sc = _edge_scores(a, bb, src, dst, b0, N)
    cluster, nes, ncl = _greedy_merge(src, dst, sc, active)
    new_x = jax.ops.segment_sum(x, cluster, num_segments=N) * nes[:, None]
    ns, nd = _coarsen_edges(cluster, src, dst, ncl, N, E)
    return new_x, ns, nd, (cluster, nes), jnp.arange(N, dtype=jnp.int32) < ncl


def _gcn_agg(xw, src, dst, N, b):
    valid = src < N
    s_c = jnp.minimum(src, N - 1)
    d_c = jnp.minimum(dst, N - 1)
    deg = jnp.ones((N,), xw.dtype).at[jnp.where(valid, d_c, N)].add(1.0)
    dinv = lax.rsqrt(deg)
    xws = xw * dinv[:, None]
    agg = xws.at[jnp.where(valid, d_c, N)].add(xws[s_c])
    return agg * dinv[:, None] + b


def kernel(x, edge_index, W1, b1, p1w, p1b, p2w, p2b, p3w, p3b, W2, b2, W3, b3, W4, b4, fc1w, fc1b, fc2w, fc2b):
    N = x.shape[0]
    F = x.shape[1]
    src = edge_index[:, 0].astype(jnp.int32)
    dst = edge_index[:, 1].astype(jnp.int32)
    C = W1.shape[1] + F

    x_in = x
    xw1 = pallas_mm(x, W1, jnp.zeros_like(b1))
    h = jax.nn.relu(_gcn_agg(xw1, src, dst, N, b1))
    xx = jnp.concatenate([x_in, h], axis=-1)
    act = jnp.ones((N,), bool)

    def proj2(xc, pw):
        w2 = jnp.concatenate([pw[:C], pw[C:]], axis=1)
        ab = pallas_mm(xc, w2, jnp.zeros((2,), jnp.float32))
        return ab[:, 0], ab[:, 1]

    a1, bb1 = proj2(xx, p1w)
    xx, s1, d1, u1, act = _edge_pool(xx, a1, bb1, src, dst, p1b[0], act)
    a2, bb2 = proj2(xx, p2w)
    xx, s2, d2, u2, act = _edge_pool(xx, a2, bb2, s1, d1, p2b[0], act)
    a3, bb3 = proj2(xx, p3w)
    xx, s3, d3, u3, act = _edge_pool(xx, a3, bb3, s2, d2, p3b[0], act)

    xp3 = xx
    hh = jax.nn.relu(_gcn_agg(pallas_mm(xx, W2, jnp.zeros_like(b2)), s3, d3, N, b2))
    xw3 = pallas_mm(xp3, W3[:C], jnp.zeros_like(b3)) + pallas_mm(hh, W3[C:], jnp.zeros_like(b3))
    hh = jax.nn.relu(_gcn_agg(xw3, s3, d3, N, b3))
    xw4 = pallas_mm(xp3, W4[:C], jnp.zeros_like(b4)) + pallas_mm(hh, W4[C:], jnp.zeros_like(b4))
    hh = jax.nn.relu(_gcn_agg(xw4, s3, d3, N, b4))

    c1, nes1 = u1
    c2, nes2 = u2
    c3, nes3 = u3
    i2 = c2[c1]
    idx = c3[i2]
    scale = 1.0 / (nes1[c1] * nes2[i2] * nes3[idx])
    hh = hh[idx] * scale[:, None]

    h2 = jax.nn.relu(
        pallas_mm(x_in, fc1w[:F], fc1b) + pallas_mm(hh, fc1w[F:], jnp.zeros_like(fc1b))
    )
    out = pallas_mm(x_in, fc2w[:F], fc2b) + pallas_mm(h2, fc2w[F:], jnp.zeros_like(fc2b))
    return jax.nn.sigmoid(out).reshape(-1)
